# baseline (speedup 1.0000x reference)
import functools

import jax
import jax.numpy as jnp
from jax.experimental import pallas as pl
from jax.experimental.pallas import tpu as pltpu

T, N, F, H, L = 4, 100000, 2, 16, 8
DIN = F + L + H + 1

_BLK = 2000


def _gate_body(r_ref, u_ref, h_ref, reset_o, update_o, rh_o):
    r = jax.nn.sigmoid(r_ref[...])
    u = jax.nn.sigmoid(u_ref[...])
    h = h_ref[...]
    reset_o[...] = r
    update_o[...] = u
    rh_o[...] = r * h


def _gates(r_raw, u_raw, hidden):
    grid = (N // _BLK,)
    spec = pl.BlockSpec((_BLK, H), lambda i: (i, 0))
    return pl.pallas_call(
        _gate_body,
        grid=grid,
        in_specs=[spec, spec, spec],
        out_specs=[spec, spec, spec],
        out_shape=[jax.ShapeDtypeStruct((N, H), jnp.float32)] * 3,
    )(r_raw, u_raw, hidden)


def _hid_body(c_ref, u_ref, h_ref, h_o):
    c = jnp.tanh(c_ref[...])
    u = u_ref[...]
    h_o[...] = u * h_ref[...] + (1.0 - u) * c


def _hidden_update(c_raw, update, hidden):
    grid = (N // _BLK,)
    spec = pl.BlockSpec((_BLK, H), lambda i: (i, 0))
    return pl.pallas_call(
        _hid_body,
        grid=grid,
        in_specs=[spec, spec, spec],
        out_specs=spec,
        out_shape=jax.ShapeDtypeStruct((N, H), jnp.float32),
    )(c_raw, update, hidden)


def _gatv2(xin, src, dst, ew, Wl, Wr, att, b):
    xl = xin @ Wl
    xr = xin @ Wr
    e = jax.nn.leaky_relu(xl[src] + xr[dst], negative_slope=0.2) @ att
    m = jax.ops.segment_max(e, dst, num_segments=N)
    m = jnp.where(jnp.isfinite(m), m, 0.0)
    ex = jnp.exp(e - m[dst])
    denom = jax.ops.segment_sum(ex, dst, num_segments=N)
    alpha = ex / (denom[dst] + 1e-16)
    msg = xl[src] * (alpha * ew)[:, None]
    return jax.ops.segment_sum(msg, dst, num_segments=N) + b


def kernel(x, edge_index, mask, labels, edge_weight,
           reset_Wl, reset_Wr, reset_att, reset_b,
           update_Wl, update_Wr, update_att, update_b,
           cell_Wl, cell_Wr, cell_att, cell_b,
           final_Wl, final_Wr, final_att, final_b,
           W1, b1, W2, b2):
    src = edge_index[0]
    dst = edge_index[1]
    mask_b = mask[:, None]
    mask_f = mask_b.astype(jnp.float32)
    hidden = jnp.ones((N, H), dtype=jnp.float32)
    masked_input = jnp.where(mask_b[None, :, :], x, 0.0)
    x2 = masked_input[0]
    for t in range(T):
        ft = jnp.concatenate([x2, mask_f, labels], axis=1)
        hin = jnp.concatenate([ft, hidden], axis=1)
        r_raw = _gatv2(hin, src, dst, edge_weight, reset_Wl, reset_Wr, reset_att, reset_b)
        u_raw = _gatv2(hin, src, dst, edge_weight, update_Wl, update_Wr, update_att, update_b)
        reset, update, rh = _gates(r_raw, u_raw, hidden)
        cin = jnp.concatenate([ft, rh], axis=1)
        c_raw = _gatv2(cin, src, dst, edge_weight, cell_Wl, cell_Wr, cell_att, cell_b)
        hidden = _hidden_update(c_raw, update, hidden)
        y1 = hidden @ W1 + b1
        x1 = jnp.where(mask_b, masked_input[t], y1)
        ft2 = jnp.concatenate([x1, mask_f, hidden, labels], axis=1)
        s = jax.nn.relu(_gatv2(ft2, src, dst, edge_weight, final_Wl, final_Wr, final_att, final_b))
        y2 = jnp.concatenate([s, hidden], axis=1) @ W2 + b2
        x2 = jnp.where(mask_b, masked_input[t], y2)
        masked_input = masked_input.at[t].set(x2)
    return masked_input


# speedup vs baseline: 35.8079x; 35.8079x over previous
import functools

import jax
import jax.numpy as jnp
from jax import lax
from jax.experimental import pallas as pl
from jax.experimental.pallas import tpu as pltpu
from jax.experimental.pallas import tpu_sc as plsc

T, N, F, H, L = 4, 100000, 2, 16, 8
DIN = F + L + H + 1
E = 3200000

NC, NS = 2, 16
NW = NC * NS
EPW = E // NW
C = 400
NCH = EPW // C
G = C // 16
NP = 100096
RPT = NP // NS

_BLK = 2000
_mesh = plsc.VectorSubcoreMesh(core_axis_name="c", subcore_axis_name="s")


@functools.partial(
    pl.kernel,
    out_type=[
        jax.ShapeDtypeStruct((NC * NP, H), jnp.float32),
        jax.ShapeDtypeStruct((NC * NP,), jnp.float32),
    ],
    mesh=_mesh,
    compiler_params=pltpu.CompilerParams(
        use_tc_tiling_on_sc=False, needs_layout_passes=False),
    scratch_types=[
        pltpu.VMEM_SHARED((NP, H), jnp.float32),
        pltpu.VMEM_SHARED((NP,), jnp.float32),
        pltpu.VMEM((C,), jnp.int32),
        pltpu.VMEM((C,), jnp.int32),
        pltpu.VMEM((C,), jnp.float32),
        pltpu.VMEM((C, H), jnp.float32),
        pltpu.VMEM((C, H), jnp.float32),
        pltpu.VMEM((C,), jnp.float32),
        pltpu.VMEM((C,), jnp.float32),
        pltpu.VMEM((16,), jnp.float32),
        pltpu.SemaphoreType.DMA,
        pltpu.SemaphoreType.DMA,
    ],
)
def _sc_conv(xl_hbm, xr_hbm, src_hbm, dst_hbm, ew_hbm, att_hbm,
             acc_out, den_out,
             acc_sh, den_sh, src_v, dst_v, ew_v, xlr_v, xrr_v, ex_v, w_v,
             att_v, sem1, sem2):
    c = lax.axis_index("c")
    s = lax.axis_index("s")
    wid = s * NC + c
    row0 = s * RPT

    def _z2(i, _):
        xlr_v[i, :] = jnp.zeros((16,), jnp.float32)
        return 0
    lax.fori_loop(0, C, _z2, 0)

    def _z1(i, _):
        ew_v[pl.ds(i * 16, 16)] = jnp.zeros((16,), jnp.float32)
        return 0
    lax.fori_loop(0, C // 16, _z1, 0)

    for j in range(15):
        pltpu.sync_copy(xlr_v, acc_sh.at[pl.ds(row0 + j * C, C)])
        pltpu.sync_copy(ew_v, den_sh.at[pl.ds(row0 + j * C, C)])
    pltpu.sync_copy(xlr_v.at[pl.ds(0, RPT - 15 * C)],
                    acc_sh.at[pl.ds(row0 + 15 * C, RPT - 15 * C)])
    pltpu.sync_copy(ew_v.at[pl.ds(0, RPT - 15 * C)],
                    den_sh.at[pl.ds(row0 + 15 * C, RPT - 15 * C)])
    pltpu.sync_copy(att_hbm, att_v)
    plsc.subcore_barrier()

    att_vec = att_v[...]
    base = wid * EPW

    def chunk_body(k, _):
        off = base + k * C
        pltpu.sync_copy(src_hbm.at[pl.ds(off, C)], src_v)
        pltpu.sync_copy(dst_hbm.at[pl.ds(off, C)], dst_v)
        pltpu.sync_copy(ew_hbm.at[pl.ds(off, C)], ew_v)
        cp1 = pltpu.async_copy(xl_hbm.at[src_v], xlr_v, sem1)
        cp2 = pltpu.async_copy(xr_hbm.at[dst_v], xrr_v, sem2)
        cp1.wait()
        cp2.wait()

        def group_body(g, _):
            gb = g * 16
            ids = lax.iota(jnp.int32, 16) + gb
            acc_e = jnp.zeros((16,), jnp.float32)
            for h in range(16):
                hh = jnp.full((16,), h, jnp.int32)
                a = plsc.load_gather(xlr_v, [ids, hh])
                b = plsc.load_gather(xrr_v, [ids, hh])
                sv = a + b
                lv = jnp.maximum(sv, 0.0) + 0.2 * jnp.minimum(sv, 0.0)
                ah = plsc.load_gather(att_v, [hh])
                acc_e = acc_e + ah * lv
            exv = jnp.exp(acc_e)
            ex_v[pl.ds(gb, 16)] = exv
            wv = exv * ew_v[pl.ds(gb, 16)]
            w_v[pl.ds(gb, 16)] = wv
            lane = lax.iota(jnp.int32, 16)
            for e in range(16):
                ee = jnp.full((16,), gb + e, jnp.int32)
                wbc = plsc.load_gather(w_v, [ee])
                row = plsc.load_gather(xlr_v, [ee, lane])
                plsc.store_scatter(xlr_v, [ee, lane], row * wbc)
            return 0

        lax.fori_loop(0, G, group_body, 0)
        pltpu.sync_copy(xlr_v, acc_sh.at[dst_v], add=True)
        pltpu.sync_copy(ex_v, den_sh.at[dst_v], add=True)
        return 0

    lax.fori_loop(0, NCH, chunk_body, 0)
    plsc.subcore_barrier()
    pltpu.sync_copy(acc_sh.at[pl.ds(row0, RPT)],
                    acc_out.at[pl.ds(c * NP + row0, RPT)])
    pltpu.sync_copy(den_sh.at[pl.ds(row0, RPT)],
                    den_out.at[pl.ds(c * NP + row0, RPT)])


def _proj2_body(x_ref, wla_ref, wra_ref, wlb_ref, wrb_ref,
                xla_o, xra_o, xlb_o, xrb_o):
    x = x_ref[...]
    xla_o[...] = jnp.dot(x, wla_ref[...], preferred_element_type=jnp.float32)
    xra_o[...] = jnp.dot(x, wra_ref[...], preferred_element_type=jnp.float32)
    xlb_o[...] = jnp.dot(x, wlb_ref[...], preferred_element_type=jnp.float32)
    xrb_o[...] = jnp.dot(x, wrb_ref[...], preferred_element_type=jnp.float32)


def _proj2(xin, wla, wra, wlb, wrb):
    k = xin.shape[1]
    xspec = pl.BlockSpec((_BLK, k), lambda i: (i, 0))
    wspec = pl.BlockSpec((k, H), lambda i: (0, 0))
    ospec = pl.BlockSpec((_BLK, H), lambda i: (i, 0))
    return pl.pallas_call(
        _proj2_body,
        grid=(N // _BLK,),
        in_specs=[xspec, wspec, wspec, wspec, wspec],
        out_specs=[ospec] * 4,
        out_shape=[jax.ShapeDtypeStruct((N, H), jnp.float32)] * 4,
    )(xin, wla, wra, wlb, wrb)


def _proj1_body(x_ref, wl_ref, wr_ref, xl_o, xr_o):
    x = x_ref[...]
    xl_o[...] = jnp.dot(x, wl_ref[...], preferred_element_type=jnp.float32)
    xr_o[...] = jnp.dot(x, wr_ref[...], preferred_element_type=jnp.float32)


def _proj1(xin, wl, wr):
    k = xin.shape[1]
    xspec = pl.BlockSpec((_BLK, k), lambda i: (i, 0))
    wspec = pl.BlockSpec((k, H), lambda i: (0, 0))
    ospec = pl.BlockSpec((_BLK, H), lambda i: (i, 0))
    return pl.pallas_call(
        _proj1_body,
        grid=(N // _BLK,),
        in_specs=[xspec, wspec, wspec],
        out_specs=[ospec, ospec],
        out_shape=[jax.ShapeDtypeStruct((N, H), jnp.float32)] * 2,
    )(xin, wl, wr)


def _mergevals(acc_ref, den_ref, b_ref):
    num = acc_ref[0] + acc_ref[1]
    den = den_ref[0] + den_ref[1] + 1e-16
    return num / den + b_ref[...]


def _gates_body(accr_ref, denr_ref, rb_ref, accu_ref, denu_ref, ub_ref, h_ref,
                reset_o, update_o, rh_o):
    r = jax.nn.sigmoid(_mergevals(accr_ref, denr_ref, rb_ref))
    u = jax.nn.sigmoid(_mergevals(accu_ref, denu_ref, ub_ref))
    reset_o[...] = r
    update_o[...] = u
    rh_o[...] = r * h_ref[...]


def _cell_body(accc_ref, denc_ref, cb_ref, u_ref, h_ref, h_o):
    cell = jnp.tanh(_mergevals(accc_ref, denc_ref, cb_ref))
    u = u_ref[...]
    h_o[...] = u * h_ref[...] + (1.0 - u) * cell


def _ft2_body(h_ref, w1_ref, b1_ref, m_ref, xt_ref, lab_ref, ft2_o):
    h = h_ref[...]
    y1 = jnp.dot(h, w1_ref[...], preferred_element_type=jnp.float32) + b1_ref[...]
    m = m_ref[...] > 0.0
    x1 = jnp.where(m, xt_ref[...], y1)
    ft2_o[...] = jnp.concatenate([x1, m_ref[...], h, lab_ref[...]], axis=1)


def _final_body(accf_ref, denf_ref, fb_ref, h_ref, w2_ref, b2_ref, m_ref,
                xt_ref, x2_o):
    sval = jnp.maximum(_mergevals(accf_ref, denf_ref, fb_ref), 0.0)
    h = h_ref[...]
    y2 = (jnp.dot(sval, w2_ref[: H], preferred_element_type=jnp.float32)
          + jnp.dot(h, w2_ref[H:], preferred_element_type=jnp.float32)
          + b2_ref[...])
    x2_o[...] = jnp.where(m_ref[...] > 0.0, xt_ref[...], y2)


def _acc_spec():
    return pl.BlockSpec((NC, _BLK, H), lambda i: (0, i, 0))


def _den_spec():
    return pl.BlockSpec((NC, _BLK, 1), lambda i: (0, i, 0))


def _row_spec(w):
    return pl.BlockSpec((_BLK, w), lambda i: (i, 0))


def _full_spec(a, b):
    return pl.BlockSpec((a, b), lambda i: (0, 0))


def _gates_merge(accr, denr, rb, accu, denu, ub, hidden):
    denr, denu = denr[:, :, None], denu[:, :, None]
    return pl.pallas_call(
        _gates_body,
        grid=(N // _BLK,),
        in_specs=[_acc_spec(), _den_spec(), _full_spec(1, H),
                  _acc_spec(), _den_spec(), _full_spec(1, H), _row_spec(H)],
        out_specs=[_row_spec(H)] * 3,
        out_shape=[jax.ShapeDtypeStruct((N, H), jnp.float32)] * 3,
    )(accr, denr, rb, accu, denu, ub, hidden)


def _cell_merge(accc, denc, cb, update, hidden):
    denc = denc[:, :, None]
    return pl.pallas_call(
        _cell_body,
        grid=(N // _BLK,),
        in_specs=[_acc_spec(), _den_spec(), _full_spec(1, H),
                  _row_spec(H), _row_spec(H)],
        out_specs=_row_spec(H),
        out_shape=jax.ShapeDtypeStruct((N, H), jnp.float32),
    )(accc, denc, cb, update, hidden)


def _ft2_build(hidden, w1, b1, maskf, xt, labels):
    return pl.pallas_call(
        _ft2_body,
        grid=(N // _BLK,),
        in_specs=[_row_spec(H), _full_spec(H, F), _full_spec(1, F),
                  _row_spec(1), _row_spec(F), _row_spec(L)],
        out_specs=_row_spec(DIN),
        out_shape=jax.ShapeDtypeStruct((N, DIN), jnp.float32),
    )(hidden, w1, b1, maskf, xt, labels)


def _final_merge(accf, denf, fb, hidden, w2, b2, maskf, xt):
    denf = denf[:, :, None]
    return pl.pallas_call(
        _final_body,
        grid=(N // _BLK,),
        in_specs=[_acc_spec(), _den_spec(), _full_spec(1, H), _row_spec(H),
                  _full_spec(2 * H, F), _full_spec(1, F), _row_spec(1),
                  _row_spec(F)],
        out_specs=_row_spec(F),
        out_shape=jax.ShapeDtypeStruct((N, F), jnp.float32),
    )(accf, denf, fb, hidden, w2, b2, maskf, xt)


def _conv(xl, xr, src, dst, ew, att):
    acc, den = _sc_conv(xl, xr, src, dst, ew, att)
    return acc.reshape(NC, NP, H), den.reshape(NC, NP)


def kernel(x, edge_index, mask, labels, edge_weight,
           reset_Wl, reset_Wr, reset_att, reset_b,
           update_Wl, update_Wr, update_att, update_b,
           cell_Wl, cell_Wr, cell_att, cell_b,
           final_Wl, final_Wr, final_att, final_b,
           W1, b1, W2, b2):
    src = edge_index[0]
    dst = edge_index[1]
    mask_b = mask[:, None]
    mask_f = mask_b.astype(jnp.float32)
    hidden = jnp.ones((N, H), dtype=jnp.float32)
    xm = jnp.where(mask_b[None, :, :], x, 0.0)
    x2 = xm[0]
    rb, ub = reset_b[None, :], update_b[None, :]
    cb, fb = cell_b[None, :], final_b[None, :]
    b1r, b2r = b1[None, :], b2[None, :]
    outs = []
    for t in range(T):
        ft = jnp.concatenate([x2, mask_f, labels], axis=1)
        hin = jnp.concatenate([ft, hidden], axis=1)
        xl_r, xr_r, xl_u, xr_u = _proj2(hin, reset_Wl, reset_Wr,
                                        update_Wl, update_Wr)
        acc_r, den_r = _conv(xl_r, xr_r, src, dst, edge_weight, reset_att)
        acc_u, den_u = _conv(xl_u, xr_u, src, dst, edge_weight, update_att)
        reset, update, rh = _gates_merge(acc_r, den_r, rb, acc_u, den_u, ub,
                                         hidden)
        cin = jnp.concatenate([ft, rh], axis=1)
        xl_c, xr_c = _proj1(cin, cell_Wl, cell_Wr)
        acc_c, den_c = _conv(xl_c, xr_c, src, dst, edge_weight, cell_att)
        hidden = _cell_merge(acc_c, den_c, cb, update, hidden)
        ft2 = _ft2_build(hidden, W1, b1r, mask_f, xm[t], labels)
        xl_f, xr_f = _proj1(ft2, final_Wl, final_Wr)
        acc_f, den_f = _conv(xl_f, xr_f, src, dst, edge_weight, final_att)
        x2 = _final_merge(acc_f, den_f, fb, hidden, W2, b2r, mask_f, xm[t])
        outs.append(x2)
    return jnp.stack(outs)


# speedup vs baseline: 60.7018x; 1.6952x over previous
import functools

import jax
import jax.numpy as jnp
from jax import lax
from jax.experimental import pallas as pl
from jax.experimental.pallas import tpu as pltpu
from jax.experimental.pallas import tpu_sc as plsc

T, N, F, H, L = 4, 100000, 2, 16, 8
DIN = F + L + H + 1
E = 3200000

NC, NS = 2, 16
NW = NC * NS
EPW = E // NW
C = 400
NCH = EPW // C
G = C // 16
NP = 100096
RPT = NP // NS

_BLK = 2000
_mesh = plsc.VectorSubcoreMesh(core_axis_name="c", subcore_axis_name="s")


SL = 5
W = C // SL


@functools.partial(
    pl.kernel,
    out_type=[
        jax.ShapeDtypeStruct((NC * NP, H), jnp.float32),
        jax.ShapeDtypeStruct((NC * NP,), jnp.float32),
    ],
    mesh=_mesh,
    compiler_params=pltpu.CompilerParams(
        use_tc_tiling_on_sc=False, needs_layout_passes=False),
    scratch_types=[
        pltpu.VMEM_SHARED((NP, H), jnp.float32),
        pltpu.VMEM_SHARED((NP,), jnp.float32),
        pltpu.VMEM((2, SL, W), jnp.int32),
        pltpu.VMEM((2, SL, W), jnp.int32),
        pltpu.VMEM((2, C), jnp.float32),
        pltpu.VMEM((SL, W, H), jnp.float32),
        pltpu.VMEM((SL, W, H), jnp.float32),
        pltpu.VMEM((C,), jnp.float32),
        pltpu.VMEM((16,), jnp.float32),
        pltpu.SemaphoreType.DMA((SL,)),
        pltpu.SemaphoreType.DMA((SL,)),
        pltpu.SemaphoreType.DMA,
        pltpu.SemaphoreType.DMA,
    ],
)
def _sc_conv(xl_hbm, xr_hbm, src_hbm, dst_hbm, ew_hbm, att_hbm,
             acc_out, den_out,
             acc_sh, den_sh, src3, dst3, ew2, xlr3, xrr3, exf, att_v,
             semg1, semg2, sem_l, sem_s):
    c = lax.axis_index("c")
    s = lax.axis_index("s")
    wid = s * NC + c
    row0 = s * RPT
    base = wid * EPW
    lane = lax.iota(jnp.int32, 16)

    def _zx(i, _):
        xlr3[0, i, :] = jnp.zeros((16,), jnp.float32)
        return 0
    lax.fori_loop(0, W, _zx, 0)

    def _ze(i, _):
        exf[pl.ds(i * 16, 16)] = jnp.zeros((16,), jnp.float32)
        return 0
    lax.fori_loop(0, C // 16, _ze, 0)

    def _za(i, _):
        pltpu.sync_copy(xlr3.at[0], acc_sh.at[pl.ds(row0 + i * W, W)])
        return 0
    lax.fori_loop(0, RPT // W, _za, 0)
    pltpu.sync_copy(xlr3.at[0, pl.ds(0, RPT - (RPT // W) * W)],
                    acc_sh.at[pl.ds(row0 + (RPT // W) * W, RPT - (RPT // W) * W)])

    def _zd(i, _):
        pltpu.sync_copy(exf, den_sh.at[pl.ds(row0 + i * C, C)])
        return 0
    lax.fori_loop(0, RPT // C, _zd, 0)
    pltpu.sync_copy(exf.at[pl.ds(0, RPT - (RPT // C) * C)],
                    den_sh.at[pl.ds(row0 + (RPT // C) * C, RPT - (RPT // C) * C)])

    pltpu.sync_copy(att_hbm, att_v)
    plsc.subcore_barrier()

    attb = [plsc.load_gather(att_v, [jnp.full((16,), h, jnp.int32)])
            for h in range(16)]

    for j in range(SL):
        pltpu.sync_copy(src_hbm.at[pl.ds(base + j * W, W)], src3.at[0, j])
        pltpu.sync_copy(dst_hbm.at[pl.ds(base + j * W, W)], dst3.at[0, j])
    pltpu.sync_copy(ew_hbm.at[pl.ds(base, C)], ew2.at[0])
    for j in range(SL):
        pltpu.async_copy(xl_hbm.at[src3.at[0, j]], xlr3.at[j], semg1.at[j])
        pltpu.async_copy(xr_hbm.at[dst3.at[0, j]], xrr3.at[j], semg2.at[j])

    def sub_body(k, p):
        np_ = 1 - p

        @pl.when(k + 1 < NCH)
        def _prefetch_idx():
            off2 = base + (k + 1) * C
            for j in range(SL):
                pltpu.async_copy(src_hbm.at[pl.ds(off2 + j * W, W)],
                                 src3.at[np_, j], sem_l)
                pltpu.async_copy(dst_hbm.at[pl.ds(off2 + j * W, W)],
                                 dst3.at[np_, j], sem_l)
            pltpu.async_copy(ew_hbm.at[pl.ds(off2, C)], ew2.at[np_], sem_l)

        sdescs = []
        for j in range(SL):
            pltpu.make_async_copy(xl_hbm.at[src3.at[p, j]], xlr3.at[j],
                                  semg1.at[j]).wait()
            pltpu.make_async_copy(xr_hbm.at[dst3.at[p, j]], xrr3.at[j],
                                  semg2.at[j]).wait()
            jj = jnp.full((16,), j, jnp.int32)

            def group_body(g, _):
                gb = g * 16
                ids = lane + gb
                avs = []
                acc_e = jnp.zeros((16,), jnp.float32)
                for h in range(16):
                    hh = jnp.full((16,), h, jnp.int32)
                    a = plsc.load_gather(xlr3, [jj, ids, hh])
                    b = plsc.load_gather(xrr3, [jj, ids, hh])
                    sv = a + b
                    lv = jnp.maximum(sv, 0.0) + 0.2 * jnp.minimum(sv, 0.0)
                    acc_e = acc_e + attb[h] * lv
                    avs.append(a)
                exv = jnp.exp(acc_e)
                exf[pl.ds(j * W + gb, 16)] = exv
                wv = exv * ew2[p, pl.ds(j * W + gb, 16)]
                for h in range(16):
                    hh = jnp.full((16,), h, jnp.int32)
                    plsc.store_scatter(xlr3, [jj, ids, hh], wv * avs[h])
                return 0

            lax.fori_loop(0, W // 16, group_body, 0)
            sdescs.append(pltpu.async_copy(
                xlr3.at[j], acc_sh.at[dst3.at[p, j]], sem_s, add=True))
            sdescs.append(pltpu.async_copy(
                exf.at[pl.ds(j * W, W)], den_sh.at[dst3.at[p, j]], sem_s,
                add=True))
        for d in sdescs:
            d.wait()

        @pl.when(k + 1 < NCH)
        def _issue_next():
            off2 = base + (k + 1) * C
            for j in range(SL):
                pltpu.make_async_copy(src_hbm.at[pl.ds(off2 + j * W, W)],
                                      src3.at[np_, j], sem_l).wait()
                pltpu.make_async_copy(dst_hbm.at[pl.ds(off2 + j * W, W)],
                                      dst3.at[np_, j], sem_l).wait()
            pltpu.make_async_copy(ew_hbm.at[pl.ds(off2, C)], ew2.at[np_],
                                  sem_l).wait()
            for j in range(SL):
                pltpu.async_copy(xl_hbm.at[src3.at[np_, j]], xlr3.at[j],
                                 semg1.at[j])
                pltpu.async_copy(xr_hbm.at[dst3.at[np_, j]], xrr3.at[j],
                                 semg2.at[j])

    def pair_body(k2, _):
        sub_body(k2 * 2, 0)
        sub_body(k2 * 2 + 1, 1)
        return 0

    lax.fori_loop(0, NCH // 2, pair_body, 0)
    plsc.subcore_barrier()
    pltpu.sync_copy(acc_sh.at[pl.ds(row0, RPT)],
                    acc_out.at[pl.ds(c * NP + row0, RPT)])
    pltpu.sync_copy(den_sh.at[pl.ds(row0, RPT)],
                    den_out.at[pl.ds(c * NP + row0, RPT)])


def _proj2_body(x_ref, wla_ref, wra_ref, wlb_ref, wrb_ref,
                xla_o, xra_o, xlb_o, xrb_o):
    x = x_ref[...]
    xla_o[...] = jnp.dot(x, wla_ref[...], preferred_element_type=jnp.float32)
    xra_o[...] = jnp.dot(x, wra_ref[...], preferred_element_type=jnp.float32)
    xlb_o[...] = jnp.dot(x, wlb_ref[...], preferred_element_type=jnp.float32)
    xrb_o[...] = jnp.dot(x, wrb_ref[...], preferred_element_type=jnp.float32)


def _proj2(xin, wla, wra, wlb, wrb):
    k = xin.shape[1]
    xspec = pl.BlockSpec((_BLK, k), lambda i: (i, 0))
    wspec = pl.BlockSpec((k, H), lambda i: (0, 0))
    ospec = pl.BlockSpec((_BLK, H), lambda i: (i, 0))
    return pl.pallas_call(
        _proj2_body,
        grid=(N // _BLK,),
        in_specs=[xspec, wspec, wspec, wspec, wspec],
        out_specs=[ospec] * 4,
        out_shape=[jax.ShapeDtypeStruct((N, H), jnp.float32)] * 4,
    )(xin, wla, wra, wlb, wrb)


def _proj1_body(x_ref, wl_ref, wr_ref, xl_o, xr_o):
    x = x_ref[...]
    xl_o[...] = jnp.dot(x, wl_ref[...], preferred_element_type=jnp.float32)
    xr_o[...] = jnp.dot(x, wr_ref[...], preferred_element_type=jnp.float32)


def _proj1(xin, wl, wr):
    k = xin.shape[1]
    xspec = pl.BlockSpec((_BLK, k), lambda i: (i, 0))
    wspec = pl.BlockSpec((k, H), lambda i: (0, 0))
    ospec = pl.BlockSpec((_BLK, H), lambda i: (i, 0))
    return pl.pallas_call(
        _proj1_body,
        grid=(N // _BLK,),
        in_specs=[xspec, wspec, wspec],
        out_specs=[ospec, ospec],
        out_shape=[jax.ShapeDtypeStruct((N, H), jnp.float32)] * 2,
    )(xin, wl, wr)


def _mergevals(acc_ref, den_ref, b_ref):
    num = acc_ref[0] + acc_ref[1]
    den = den_ref[0] + den_ref[1] + 1e-16
    return num / den + b_ref[...]


def _gates_body(accr_ref, denr_ref, rb_ref, accu_ref, denu_ref, ub_ref, h_ref,
                reset_o, update_o, rh_o):
    r = jax.nn.sigmoid(_mergevals(accr_ref, denr_ref, rb_ref))
    u = jax.nn.sigmoid(_mergevals(accu_ref, denu_ref, ub_ref))
    reset_o[...] = r
    update_o[...] = u
    rh_o[...] = r * h_ref[...]


def _cell_body(accc_ref, denc_ref, cb_ref, u_ref, h_ref, h_o):
    cell = jnp.tanh(_mergevals(accc_ref, denc_ref, cb_ref))
    u = u_ref[...]
    h_o[...] = u * h_ref[...] + (1.0 - u) * cell


def _ft2_body(h_ref, w1_ref, b1_ref, m_ref, xt_ref, lab_ref, ft2_o):
    h = h_ref[...]
    y1 = jnp.dot(h, w1_ref[...], preferred_element_type=jnp.float32) + b1_ref[...]
    m = m_ref[...] > 0.0
    x1 = jnp.where(m, xt_ref[...], y1)
    ft2_o[...] = jnp.concatenate([x1, m_ref[...], h, lab_ref[...]], axis=1)


def _final_body(accf_ref, denf_ref, fb_ref, h_ref, w2_ref, b2_ref, m_ref,
                xt_ref, x2_o):
    sval = jnp.maximum(_mergevals(accf_ref, denf_ref, fb_ref), 0.0)
    h = h_ref[...]
    y2 = (jnp.dot(sval, w2_ref[: H], preferred_element_type=jnp.float32)
          + jnp.dot(h, w2_ref[H:], preferred_element_type=jnp.float32)
          + b2_ref[...])
    x2_o[...] = jnp.where(m_ref[...] > 0.0, xt_ref[...], y2)


def _acc_spec():
    return pl.BlockSpec((NC, _BLK, H), lambda i: (0, i, 0))


def _den_spec():
    return pl.BlockSpec((NC, _BLK, 1), lambda i: (0, i, 0))


def _row_spec(w):
    return pl.BlockSpec((_BLK, w), lambda i: (i, 0))


def _full_spec(a, b):
    return pl.BlockSpec((a, b), lambda i: (0, 0))


def _gates_merge(accr, denr, rb, accu, denu, ub, hidden):
    denr, denu = denr[:, :, None], denu[:, :, None]
    return pl.pallas_call(
        _gates_body,
        grid=(N // _BLK,),
        in_specs=[_acc_spec(), _den_spec(), _full_spec(1, H),
                  _acc_spec(), _den_spec(), _full_spec(1, H), _row_spec(H)],
        out_specs=[_row_spec(H)] * 3,
        out_shape=[jax.ShapeDtypeStruct((N, H), jnp.float32)] * 3,
    )(accr, denr, rb, accu, denu, ub, hidden)


def _cell_merge(accc, denc, cb, update, hidden):
    denc = denc[:, :, None]
    return pl.pallas_call(
        _cell_body,
        grid=(N // _BLK,),
        in_specs=[_acc_spec(), _den_spec(), _full_spec(1, H),
                  _row_spec(H), _row_spec(H)],
        out_specs=_row_spec(H),
        out_shape=jax.ShapeDtypeStruct((N, H), jnp.float32),
    )(accc, denc, cb, update, hidden)


def _ft2_build(hidden, w1, b1, maskf, xt, labels):
    return pl.pallas_call(
        _ft2_body,
        grid=(N // _BLK,),
        in_specs=[_row_spec(H), _full_spec(H, F), _full_spec(1, F),
                  _row_spec(1), _row_spec(F), _row_spec(L)],
        out_specs=_row_spec(DIN),
        out_shape=jax.ShapeDtypeStruct((N, DIN), jnp.float32),
    )(hidden, w1, b1, maskf, xt, labels)


def _final_merge(accf, denf, fb, hidden, w2, b2, maskf, xt):
    denf = denf[:, :, None]
    return pl.pallas_call(
        _final_body,
        grid=(N // _BLK,),
        in_specs=[_acc_spec(), _den_spec(), _full_spec(1, H), _row_spec(H),
                  _full_spec(2 * H, F), _full_spec(1, F), _row_spec(1),
                  _row_spec(F)],
        out_specs=_row_spec(F),
        out_shape=jax.ShapeDtypeStruct((N, F), jnp.float32),
    )(accf, denf, fb, hidden, w2, b2, maskf, xt)


def _conv(xl, xr, src, dst, ew, att):
    acc, den = _sc_conv(xl, xr, src, dst, ew, att)
    return acc.reshape(NC, NP, H), den.reshape(NC, NP)


def kernel(x, edge_index, mask, labels, edge_weight,
           reset_Wl, reset_Wr, reset_att, reset_b,
           update_Wl, update_Wr, update_att, update_b,
           cell_Wl, cell_Wr, cell_att, cell_b,
           final_Wl, final_Wr, final_att, final_b,
           W1, b1, W2, b2):
    src = edge_index[0]
    dst = edge_index[1]
    mask_b = mask[:, None]
    mask_f = mask_b.astype(jnp.float32)
    hidden = jnp.ones((N, H), dtype=jnp.float32)
    xm = jnp.where(mask_b[None, :, :], x, 0.0)
    x2 = xm[0]
    rb, ub = reset_b[None, :], update_b[None, :]
    cb, fb = cell_b[None, :], final_b[None, :]
    b1r, b2r = b1[None, :], b2[None, :]
    outs = []
    for t in range(T):
        ft = jnp.concatenate([x2, mask_f, labels], axis=1)
        hin = jnp.concatenate([ft, hidden], axis=1)
        xl_r, xr_r, xl_u, xr_u = _proj2(hin, reset_Wl, reset_Wr,
                                        update_Wl, update_Wr)
        acc_r, den_r = _conv(xl_r, xr_r, src, dst, edge_weight, reset_att)
        acc_u, den_u = _conv(xl_u, xr_u, src, dst, edge_weight, update_att)
        reset, update, rh = _gates_merge(acc_r, den_r, rb, acc_u, den_u, ub,
                                         hidden)
        cin = jnp.concatenate([ft, rh], axis=1)
        xl_c, xr_c = _proj1(cin, cell_Wl, cell_Wr)
        acc_c, den_c = _conv(xl_c, xr_c, src, dst, edge_weight, cell_att)
        hidden = _cell_merge(acc_c, den_c, cb, update, hidden)
        ft2 = _ft2_build(hidden, W1, b1r, mask_f, xm[t], labels)
        xl_f, xr_f = _proj1(ft2, final_Wl, final_Wr)
        acc_f, den_f = _conv(xl_f, xr_f, src, dst, edge_weight, final_att)
        x2 = _final_merge(acc_f, den_f, fb, hidden, W2, b2r, mask_f, xm[t])
        outs.append(x2)
    return jnp.stack(outs)


# speedup vs baseline: 62.9998x; 1.0379x over previous
import functools

import jax
import jax.numpy as jnp
from jax import lax
from jax.experimental import pallas as pl
from jax.experimental.pallas import tpu as pltpu
from jax.experimental.pallas import tpu_sc as plsc

T, N, F, H, L = 4, 100000, 2, 16, 8
DIN = F + L + H + 1
E = 3200000

NC, NS = 2, 16
NW = NC * NS
EPW = E // NW
C = 400
NCH = EPW // C
G = C // 16
NP = 100096
RPT = NP // NS

_BLK = 2000
_mesh = plsc.VectorSubcoreMesh(core_axis_name="c", subcore_axis_name="s")


SL = 5
W = C // SL


@functools.partial(
    pl.kernel,
    out_type=[
        jax.ShapeDtypeStruct((NC * NP, H), jnp.float32),
        jax.ShapeDtypeStruct((NC * NP,), jnp.float32),
    ],
    mesh=_mesh,
    compiler_params=pltpu.CompilerParams(
        use_tc_tiling_on_sc=False, needs_layout_passes=False),
    scratch_types=[
        pltpu.VMEM_SHARED((NP, H), jnp.float32),
        pltpu.VMEM_SHARED((NP,), jnp.float32),
        pltpu.VMEM((2, SL, W), jnp.int32),
        pltpu.VMEM((2, SL, W), jnp.int32),
        pltpu.VMEM((2, C), jnp.float32),
        pltpu.VMEM((SL, W, H), jnp.float32),
        pltpu.VMEM((SL, W, H), jnp.float32),
        pltpu.VMEM((C,), jnp.float32),
        pltpu.VMEM((16,), jnp.float32),
        pltpu.SemaphoreType.DMA((SL,)),
        pltpu.SemaphoreType.DMA((SL,)),
        pltpu.SemaphoreType.DMA,
        pltpu.SemaphoreType.DMA,
    ],
)
def _sc_conv(xl_hbm, xr_hbm, src_hbm, dst_hbm, ew_hbm, att_hbm,
             acc_out, den_out,
             acc_sh, den_sh, src3, dst3, ew2, xlr3, xrr3, exf, att_v,
             semg1, semg2, sem_l, sem_s):
    c = lax.axis_index("c")
    s = lax.axis_index("s")
    wid = s * NC + c
    row0 = s * RPT
    base = wid * EPW
    lane = lax.iota(jnp.int32, 16)

    def _zx(i, _):
        xlr3[0, i, :] = jnp.zeros((16,), jnp.float32)
        return 0
    lax.fori_loop(0, W, _zx, 0)

    def _ze(i, _):
        exf[pl.ds(i * 16, 16)] = jnp.zeros((16,), jnp.float32)
        return 0
    lax.fori_loop(0, C // 16, _ze, 0)

    def _za(i, _):
        pltpu.sync_copy(xlr3.at[0], acc_sh.at[pl.ds(row0 + i * W, W)])
        return 0
    lax.fori_loop(0, RPT // W, _za, 0)
    pltpu.sync_copy(xlr3.at[0, pl.ds(0, RPT - (RPT // W) * W)],
                    acc_sh.at[pl.ds(row0 + (RPT // W) * W, RPT - (RPT // W) * W)])

    def _zd(i, _):
        pltpu.sync_copy(exf, den_sh.at[pl.ds(row0 + i * C, C)])
        return 0
    lax.fori_loop(0, RPT // C, _zd, 0)
    pltpu.sync_copy(exf.at[pl.ds(0, RPT - (RPT // C) * C)],
                    den_sh.at[pl.ds(row0 + (RPT // C) * C, RPT - (RPT // C) * C)])

    pltpu.sync_copy(att_hbm, att_v)
    plsc.subcore_barrier()

    attb = [plsc.load_gather(att_v, [jnp.full((16,), h, jnp.int32)])
            for h in range(16)]

    for j in range(SL):
        pltpu.sync_copy(src_hbm.at[pl.ds(base + j * W, W)], src3.at[0, j])
        pltpu.sync_copy(dst_hbm.at[pl.ds(base + j * W, W)], dst3.at[0, j])
    pltpu.sync_copy(ew_hbm.at[pl.ds(base, C)], ew2.at[0])
    for j in range(SL):
        pltpu.async_copy(xl_hbm.at[src3.at[0, j]], xlr3.at[j], semg1.at[j])
        pltpu.async_copy(xr_hbm.at[dst3.at[0, j]], xrr3.at[j], semg2.at[j])

    def sub_body(k, p):
        np_ = 1 - p

        @pl.when(k + 1 < NCH)
        def _prefetch_idx():
            off2 = base + (k + 1) * C
            for j in range(SL):
                pltpu.async_copy(src_hbm.at[pl.ds(off2 + j * W, W)],
                                 src3.at[np_, j], sem_l)
                pltpu.async_copy(dst_hbm.at[pl.ds(off2 + j * W, W)],
                                 dst3.at[np_, j], sem_l)
            pltpu.async_copy(ew_hbm.at[pl.ds(off2, C)], ew2.at[np_], sem_l)

        sdescs = []
        for j in range(SL):
            pltpu.make_async_copy(xl_hbm.at[src3.at[p, j]], xlr3.at[j],
                                  semg1.at[j]).wait()
            pltpu.make_async_copy(xr_hbm.at[dst3.at[p, j]], xrr3.at[j],
                                  semg2.at[j]).wait()
            jj = jnp.full((16,), j, jnp.int32)

            def group_body(g, _):
                gb = g * 16
                ids = lane + gb
                avs = []
                acc_e = jnp.zeros((16,), jnp.float32)
                for h in range(16):
                    hh = jnp.full((16,), h, jnp.int32)
                    a = plsc.load_gather(xlr3, [jj, ids, hh])
                    b = plsc.load_gather(xrr3, [jj, ids, hh])
                    sv = a + b
                    lv = jnp.maximum(sv, 0.0) + 0.2 * jnp.minimum(sv, 0.0)
                    acc_e = acc_e + attb[h] * lv
                    avs.append(a)
                exv = jnp.exp(acc_e)
                exf[pl.ds(j * W + gb, 16)] = exv
                wv = exv * ew2[p, pl.ds(j * W + gb, 16)]
                for h in range(16):
                    hh = jnp.full((16,), h, jnp.int32)
                    plsc.store_scatter(xlr3, [jj, ids, hh], wv * avs[h])
                return 0

            lax.fori_loop(0, W // 16, group_body, 0)
            sdescs.append(pltpu.async_copy(
                xlr3.at[j], acc_sh.at[dst3.at[p, j]], sem_s, add=True))
            sdescs.append(pltpu.async_copy(
                exf.at[pl.ds(j * W, W)], den_sh.at[dst3.at[p, j]], sem_s,
                add=True))
        for d in sdescs:
            d.wait()

        @pl.when(k + 1 < NCH)
        def _issue_next():
            off2 = base + (k + 1) * C
            for j in range(SL):
                pltpu.make_async_copy(src_hbm.at[pl.ds(off2 + j * W, W)],
                                      src3.at[np_, j], sem_l).wait()
                pltpu.make_async_copy(dst_hbm.at[pl.ds(off2 + j * W, W)],
                                      dst3.at[np_, j], sem_l).wait()
            pltpu.make_async_copy(ew_hbm.at[pl.ds(off2, C)], ew2.at[np_],
                                  sem_l).wait()
            for j in range(SL):
                pltpu.async_copy(xl_hbm.at[src3.at[np_, j]], xlr3.at[j],
                                 semg1.at[j])
                pltpu.async_copy(xr_hbm.at[dst3.at[np_, j]], xrr3.at[j],
                                 semg2.at[j])

    def pair_body(k2, _):
        sub_body(k2 * 2, 0)
        sub_body(k2 * 2 + 1, 1)
        return 0

    lax.fori_loop(0, NCH // 2, pair_body, 0)
    plsc.subcore_barrier()
    pltpu.sync_copy(acc_sh.at[pl.ds(row0, RPT)],
                    acc_out.at[pl.ds(c * NP + row0, RPT)])
    pltpu.sync_copy(den_sh.at[pl.ds(row0, RPT)],
                    den_out.at[pl.ds(c * NP + row0, RPT)])


def _mergevals(acc_ref, den_ref, b):
    num = acc_ref[0] + acc_ref[1]
    den = den_ref[0] + den_ref[1] + 1e-16
    return num / den + b


def _dot(x, w_ref):
    return jnp.dot(x, w_ref[...], preferred_element_type=jnp.float32)


def _proj_ru_body(x2_ref, m_ref, lab_ref, h_ref, rwl_ref, rwr_ref, uwl_ref,
                  uwr_ref, xlr_o, xrr_o, xlu_o, xru_o):
    hin = jnp.concatenate(
        [x2_ref[...], m_ref[...], lab_ref[...], h_ref[...]], axis=1)
    xlr_o[...] = _dot(hin, rwl_ref)
    xrr_o[...] = _dot(hin, rwr_ref)
    xlu_o[...] = _dot(hin, uwl_ref)
    xru_o[...] = _dot(hin, uwr_ref)


def _gates_cellproj_body(accr_ref, denr_ref, accu_ref, denu_ref, rb_ref,
                         ub_ref, h_ref, x2_ref, m_ref, lab_ref, cwl_ref,
                         cwr_ref, upd_o, xlc_o, xrc_o):
    r = jax.nn.sigmoid(_mergevals(accr_ref, denr_ref, rb_ref[...]))
    u = jax.nn.sigmoid(_mergevals(accu_ref, denu_ref, ub_ref[...]))
    upd_o[...] = u
    cin = jnp.concatenate(
        [x2_ref[...], m_ref[...], lab_ref[...], r * h_ref[...]], axis=1)
    xlc_o[...] = _dot(cin, cwl_ref)
    xrc_o[...] = _dot(cin, cwr_ref)


def _cell_ft2proj_body(accc_ref, denc_ref, cb_ref, u_ref, h_ref, w1_ref,
                       b1_ref, m_ref, xt_ref, lab_ref, fwl_ref, fwr_ref,
                       h2_o, xlf_o, xrf_o):
    cell = jnp.tanh(_mergevals(accc_ref, denc_ref, cb_ref[...]))
    u = u_ref[...]
    h2 = u * h_ref[...] + (1.0 - u) * cell
    h2_o[...] = h2
    y1 = _dot(h2, w1_ref) + b1_ref[...]
    m = m_ref[...]
    x1 = jnp.where(m > 0.0, xt_ref[...], y1)
    ft2 = jnp.concatenate([x1, m, h2, lab_ref[...]], axis=1)
    xlf_o[...] = _dot(ft2, fwl_ref)
    xrf_o[...] = _dot(ft2, fwr_ref)


def _final_nextproj_body(accf_ref, denf_ref, fb_ref, h_ref, w2_ref, b2_ref,
                         m_ref, xt_ref, lab_ref, rwl_ref, rwr_ref, uwl_ref,
                         uwr_ref, x2_o, xlr_o, xrr_o, xlu_o, xru_o):
    sval = jnp.maximum(_mergevals(accf_ref, denf_ref, fb_ref[...]), 0.0)
    h = h_ref[...]
    y2 = (jnp.dot(sval, w2_ref[:H], preferred_element_type=jnp.float32)
          + jnp.dot(h, w2_ref[H:], preferred_element_type=jnp.float32)
          + b2_ref[...])
    m = m_ref[...]
    x2 = jnp.where(m > 0.0, xt_ref[...], y2)
    x2_o[...] = x2
    hin = jnp.concatenate([x2, m, lab_ref[...], h], axis=1)
    xlr_o[...] = _dot(hin, rwl_ref)
    xrr_o[...] = _dot(hin, rwr_ref)
    xlu_o[...] = _dot(hin, uwl_ref)
    xru_o[...] = _dot(hin, uwr_ref)


def _acc_spec():
    return pl.BlockSpec((NC, _BLK, H), lambda i: (0, i, 0))


def _den_spec():
    return pl.BlockSpec((NC, _BLK, 1), lambda i: (0, i, 0))


def _row_spec(w):
    return pl.BlockSpec((_BLK, w), lambda i: (i, 0))


def _full_spec(a, b):
    return pl.BlockSpec((a, b), lambda i: (0, 0))


_GRID = (N // _BLK,)
_NH = jax.ShapeDtypeStruct((N, H), jnp.float32)
_NF = jax.ShapeDtypeStruct((N, F), jnp.float32)


def _proj_ru(x2, maskf, labels, hidden, rwl, rwr, uwl, uwr):
    return pl.pallas_call(
        _proj_ru_body,
        grid=_GRID,
        in_specs=[_row_spec(F), _row_spec(1), _row_spec(L), _row_spec(H)]
                 + [_full_spec(DIN, H)] * 4,
        out_specs=[_row_spec(H)] * 4,
        out_shape=[_NH] * 4,
    )(x2, maskf, labels, hidden, rwl, rwr, uwl, uwr)


def _gates_cellproj(accr, denr, accu, denu, rb, ub, hidden, x2, maskf,
                    labels, cwl, cwr):
    accr, accu = accr.reshape(NC, NP, H), accu.reshape(NC, NP, H)
    denr = denr.reshape(NC, NP, 1)
    denu = denu.reshape(NC, NP, 1)
    return pl.pallas_call(
        _gates_cellproj_body,
        grid=_GRID,
        in_specs=[_acc_spec(), _den_spec(), _acc_spec(), _den_spec(),
                  _full_spec(1, H), _full_spec(1, H), _row_spec(H),
                  _row_spec(F), _row_spec(1), _row_spec(L),
                  _full_spec(DIN, H), _full_spec(DIN, H)],
        out_specs=[_row_spec(H)] * 3,
        out_shape=[_NH] * 3,
    )(accr, denr, accu, denu, rb, ub, hidden, x2, maskf, labels, cwl, cwr)


def _cell_ft2proj(accc, denc, cb, update, hidden, w1, b1, maskf, xt, labels,
                  fwl, fwr):
    accc = accc.reshape(NC, NP, H)
    denc = denc.reshape(NC, NP, 1)
    return pl.pallas_call(
        _cell_ft2proj_body,
        grid=_GRID,
        in_specs=[_acc_spec(), _den_spec(), _full_spec(1, H), _row_spec(H),
                  _row_spec(H), _full_spec(H, F), _full_spec(1, F),
                  _row_spec(1), _row_spec(F), _row_spec(L),
                  _full_spec(DIN, H), _full_spec(DIN, H)],
        out_specs=[_row_spec(H)] * 3,
        out_shape=[_NH] * 3,
    )(accc, denc, cb, update, hidden, w1, b1, maskf, xt, labels, fwl, fwr)


def _final_nextproj(accf, denf, fb, hidden, w2, b2, maskf, xt, labels,
                    rwl, rwr, uwl, uwr):
    accf = accf.reshape(NC, NP, H)
    denf = denf.reshape(NC, NP, 1)
    return pl.pallas_call(
        _final_nextproj_body,
        grid=_GRID,
        in_specs=[_acc_spec(), _den_spec(), _full_spec(1, H), _row_spec(H),
                  _full_spec(2 * H, F), _full_spec(1, F), _row_spec(1),
                  _row_spec(F), _row_spec(L)] + [_full_spec(DIN, H)] * 4,
        out_specs=[_row_spec(F)] + [_row_spec(H)] * 4,
        out_shape=[_NF] + [_NH] * 4,
    )(accf, denf, fb, hidden, w2, b2, maskf, xt, labels, rwl, rwr, uwl, uwr)


def kernel(x, edge_index, mask, labels, edge_weight,
           reset_Wl, reset_Wr, reset_att, reset_b,
           update_Wl, update_Wr, update_att, update_b,
           cell_Wl, cell_Wr, cell_att, cell_b,
           final_Wl, final_Wr, final_att, final_b,
           W1, b1, W2, b2):
    src = edge_index[0]
    dst = edge_index[1]
    mask_b = mask[:, None]
    mask_f = mask_b.astype(jnp.float32)
    hidden = jnp.ones((N, H), dtype=jnp.float32)
    xm = jnp.where(mask_b[None, :, :], x, 0.0)
    x2 = xm[0]
    rb, ub = reset_b[None, :], update_b[None, :]
    cb, fb = cell_b[None, :], final_b[None, :]
    b1r, b2r = b1[None, :], b2[None, :]
    xl_r, xr_r, xl_u, xr_u = _proj_ru(x2, mask_f, labels, hidden,
                                      reset_Wl, reset_Wr, update_Wl, update_Wr)
    outs = []
    for t in range(T):
        acc_r, den_r = _sc_conv(xl_r, xr_r, src, dst, edge_weight, reset_att)
        acc_u, den_u = _sc_conv(xl_u, xr_u, src, dst, edge_weight, update_att)
        update, xl_c, xr_c = _gates_cellproj(
            acc_r, den_r, acc_u, den_u, rb, ub, hidden, x2, mask_f, labels,
            cell_Wl, cell_Wr)
        acc_c, den_c = _sc_conv(xl_c, xr_c, src, dst, edge_weight, cell_att)
        hidden, xl_f, xr_f = _cell_ft2proj(
            acc_c, den_c, cb, update, hidden, W1, b1r, mask_f, xm[t], labels,
            final_Wl, final_Wr)
        acc_f, den_f = _sc_conv(xl_f, xr_f, src, dst, edge_weight, final_att)
        x2, xl_r, xr_r, xl_u, xr_u = _final_nextproj(
            acc_f, den_f, fb, hidden, W2, b2r, mask_f, xm[t], labels,
            reset_Wl, reset_Wr, update_Wl, update_Wr)
        outs.append(x2)
    return jnp.stack(outs)


# speedup vs baseline: 63.1386x; 1.0022x over previous
import functools

import jax
import jax.numpy as jnp
from jax import lax
from jax.experimental import pallas as pl
from jax.experimental.pallas import tpu as pltpu
from jax.experimental.pallas import tpu_sc as plsc

T, N, F, H, L = 4, 100000, 2, 16, 8
DIN = F + L + H + 1
E = 3200000

NC, NS = 2, 16
NW = NC * NS
EPW = E // NW
C = 400
NCH = EPW // C
G = C // 16
NP = 100096
RPT = NP // NS

_BLK = 2000
_mesh = plsc.VectorSubcoreMesh(core_axis_name="c", subcore_axis_name="s")


SL = 5
W = C // SL


@functools.partial(
    pl.kernel,
    out_type=[
        jax.ShapeDtypeStruct((NC * NP, H), jnp.float32),
        jax.ShapeDtypeStruct((NC * NP,), jnp.float32),
    ],
    mesh=_mesh,
    compiler_params=pltpu.CompilerParams(
        use_tc_tiling_on_sc=False, needs_layout_passes=False),
    scratch_types=[
        pltpu.VMEM_SHARED((NP, H), jnp.float32),
        pltpu.VMEM_SHARED((NP,), jnp.float32),
        pltpu.VMEM((2, SL, W), jnp.int32),
        pltpu.VMEM((2, SL, W), jnp.int32),
        pltpu.VMEM((2, C), jnp.float32),
        pltpu.VMEM((SL, W, H), jnp.float32),
        pltpu.VMEM((SL, W, H), jnp.float32),
        pltpu.VMEM((C,), jnp.float32),
        pltpu.VMEM((16,), jnp.float32),
        pltpu.SemaphoreType.DMA((SL,)),
        pltpu.SemaphoreType.DMA((SL,)),
        pltpu.SemaphoreType.DMA,
        pltpu.SemaphoreType.DMA,
    ],
)
def _sc_conv(xl_hbm, xr_hbm, src_hbm, dst_hbm, ew_hbm, att_hbm,
             acc_out, den_out,
             acc_sh, den_sh, src3, dst3, ew2, xlr3, xrr3, exf, att_v,
             semg1, semg2, sem_l, sem_s):
    c = lax.axis_index("c")
    s = lax.axis_index("s")
    wid = s * NC + c
    row0 = s * RPT
    base = wid * EPW
    lane = lax.iota(jnp.int32, 16)

    def _zx(i, _):
        xlr3[0, i, :] = jnp.zeros((16,), jnp.float32)
        return 0
    lax.fori_loop(0, W, _zx, 0)

    def _ze(i, _):
        exf[pl.ds(i * 16, 16)] = jnp.zeros((16,), jnp.float32)
        return 0
    lax.fori_loop(0, C // 16, _ze, 0)

    def _za(i, _):
        pltpu.async_copy(xlr3.at[0], acc_sh.at[pl.ds(row0 + i * W, W)], sem_s)
        return 0
    lax.fori_loop(0, RPT // W, _za, 0)
    pltpu.async_copy(xlr3.at[0, pl.ds(0, RPT - (RPT // W) * W)],
                     acc_sh.at[pl.ds(row0 + (RPT // W) * W, RPT - (RPT // W) * W)],
                     sem_s)

    def _zd(i, _):
        pltpu.async_copy(exf, den_sh.at[pl.ds(row0 + i * C, C)], sem_s)
        return 0
    lax.fori_loop(0, RPT // C, _zd, 0)
    pltpu.async_copy(exf.at[pl.ds(0, RPT - (RPT // C) * C)],
                     den_sh.at[pl.ds(row0 + (RPT // C) * C, RPT - (RPT // C) * C)],
                     sem_s)

    def _zaw(i, _):
        pltpu.make_async_copy(
            xlr3.at[0], acc_sh.at[pl.ds(row0 + i * W, W)], sem_s).wait()
        return 0
    lax.fori_loop(0, RPT // W, _zaw, 0)
    pltpu.make_async_copy(
        xlr3.at[0, pl.ds(0, RPT - (RPT // W) * W)],
        acc_sh.at[pl.ds(row0 + (RPT // W) * W, RPT - (RPT // W) * W)],
        sem_s).wait()

    def _zdw(i, _):
        pltpu.make_async_copy(
            exf, den_sh.at[pl.ds(row0 + i * C, C)], sem_s).wait()
        return 0
    lax.fori_loop(0, RPT // C, _zdw, 0)
    pltpu.make_async_copy(
        exf.at[pl.ds(0, RPT - (RPT // C) * C)],
        den_sh.at[pl.ds(row0 + (RPT // C) * C, RPT - (RPT // C) * C)],
        sem_s).wait()

    pltpu.sync_copy(att_hbm, att_v)
    plsc.subcore_barrier()

    attb = [plsc.load_gather(att_v, [jnp.full((16,), h, jnp.int32)])
            for h in range(16)]

    for j in range(SL):
        pltpu.sync_copy(src_hbm.at[pl.ds(base + j * W, W)], src3.at[0, j])
        pltpu.sync_copy(dst_hbm.at[pl.ds(base + j * W, W)], dst3.at[0, j])
    pltpu.sync_copy(ew_hbm.at[pl.ds(base, C)], ew2.at[0])
    for j in range(SL):
        pltpu.async_copy(xl_hbm.at[src3.at[0, j]], xlr3.at[j], semg1.at[j])
        pltpu.async_copy(xr_hbm.at[dst3.at[0, j]], xrr3.at[j], semg2.at[j])

    def sub_body(k, p):
        np_ = 1 - p

        @pl.when(k + 1 < NCH)
        def _prefetch_idx():
            off2 = base + (k + 1) * C
            for j in range(SL):
                pltpu.async_copy(src_hbm.at[pl.ds(off2 + j * W, W)],
                                 src3.at[np_, j], sem_l)
                pltpu.async_copy(dst_hbm.at[pl.ds(off2 + j * W, W)],
                                 dst3.at[np_, j], sem_l)
            pltpu.async_copy(ew_hbm.at[pl.ds(off2, C)], ew2.at[np_], sem_l)

        sdescs = []
        for j in range(SL):
            pltpu.make_async_copy(xl_hbm.at[src3.at[p, j]], xlr3.at[j],
                                  semg1.at[j]).wait()
            pltpu.make_async_copy(xr_hbm.at[dst3.at[p, j]], xrr3.at[j],
                                  semg2.at[j]).wait()
            jj = jnp.full((16,), j, jnp.int32)

            def group_body(g, _):
                gb = g * 16
                ids = lane + gb
                avs = []
                acc_e = jnp.zeros((16,), jnp.float32)
                for h in range(16):
                    hh = jnp.full((16,), h, jnp.int32)
                    a = plsc.load_gather(xlr3, [jj, ids, hh])
                    b = plsc.load_gather(xrr3, [jj, ids, hh])
                    sv = a + b
                    lv = jnp.maximum(sv, 0.0) + 0.2 * jnp.minimum(sv, 0.0)
                    acc_e = acc_e + attb[h] * lv
                    avs.append(a)
                exv = jnp.exp(acc_e)
                exf[pl.ds(j * W + gb, 16)] = exv
                wv = exv * ew2[p, pl.ds(j * W + gb, 16)]
                for h in range(16):
                    hh = jnp.full((16,), h, jnp.int32)
                    plsc.store_scatter(xlr3, [jj, ids, hh], wv * avs[h])
                return 0

            lax.fori_loop(0, W // 16, group_body, 0)
            sdescs.append(pltpu.async_copy(
                xlr3.at[j], acc_sh.at[dst3.at[p, j]], sem_s, add=True))
            sdescs.append(pltpu.async_copy(
                exf.at[pl.ds(j * W, W)], den_sh.at[dst3.at[p, j]], sem_s,
                add=True))
        for d in sdescs:
            d.wait()

        @pl.when(k + 1 < NCH)
        def _issue_next():
            off2 = base + (k + 1) * C
            for j in range(SL):
                pltpu.make_async_copy(src_hbm.at[pl.ds(off2 + j * W, W)],
                                      src3.at[np_, j], sem_l).wait()
                pltpu.make_async_copy(dst_hbm.at[pl.ds(off2 + j * W, W)],
                                      dst3.at[np_, j], sem_l).wait()
            pltpu.make_async_copy(ew_hbm.at[pl.ds(off2, C)], ew2.at[np_],
                                  sem_l).wait()
            for j in range(SL):
                pltpu.async_copy(xl_hbm.at[src3.at[np_, j]], xlr3.at[j],
                                 semg1.at[j])
                pltpu.async_copy(xr_hbm.at[dst3.at[np_, j]], xrr3.at[j],
                                 semg2.at[j])

    def pair_body(k2, _):
        sub_body(k2 * 2, 0)
        sub_body(k2 * 2 + 1, 1)
        return 0

    lax.fori_loop(0, NCH // 2, pair_body, 0)
    plsc.subcore_barrier()
    pltpu.sync_copy(acc_sh.at[pl.ds(row0, RPT)],
                    acc_out.at[pl.ds(c * NP + row0, RPT)])
    pltpu.sync_copy(den_sh.at[pl.ds(row0, RPT)],
                    den_out.at[pl.ds(c * NP + row0, RPT)])


def _mergevals(acc_ref, den_ref, b):
    num = acc_ref[0] + acc_ref[1]
    den = den_ref[0] + den_ref[1] + 1e-16
    return num / den + b


def _dot(x, w_ref):
    return jnp.dot(x, w_ref[...], preferred_element_type=jnp.float32)


def _proj_ru_body(x2_ref, m_ref, lab_ref, h_ref, rwl_ref, rwr_ref, uwl_ref,
                  uwr_ref, xlr_o, xrr_o, xlu_o, xru_o):
    hin = jnp.concatenate(
        [x2_ref[...], m_ref[...], lab_ref[...], h_ref[...]], axis=1)
    xlr_o[...] = _dot(hin, rwl_ref)
    xrr_o[...] = _dot(hin, rwr_ref)
    xlu_o[...] = _dot(hin, uwl_ref)
    xru_o[...] = _dot(hin, uwr_ref)


def _gates_cellproj_body(accr_ref, denr_ref, accu_ref, denu_ref, rb_ref,
                         ub_ref, h_ref, x2_ref, m_ref, lab_ref, cwl_ref,
                         cwr_ref, upd_o, xlc_o, xrc_o):
    r = jax.nn.sigmoid(_mergevals(accr_ref, denr_ref, rb_ref[...]))
    u = jax.nn.sigmoid(_mergevals(accu_ref, denu_ref, ub_ref[...]))
    upd_o[...] = u
    cin = jnp.concatenate(
        [x2_ref[...], m_ref[...], lab_ref[...], r * h_ref[...]], axis=1)
    xlc_o[...] = _dot(cin, cwl_ref)
    xrc_o[...] = _dot(cin, cwr_ref)


def _cell_ft2proj_body(accc_ref, denc_ref, cb_ref, u_ref, h_ref, w1_ref,
                       b1_ref, m_ref, xt_ref, lab_ref, fwl_ref, fwr_ref,
                       h2_o, xlf_o, xrf_o):
    cell = jnp.tanh(_mergevals(accc_ref, denc_ref, cb_ref[...]))
    u = u_ref[...]
    h2 = u * h_ref[...] + (1.0 - u) * cell
    h2_o[...] = h2
    y1 = _dot(h2, w1_ref) + b1_ref[...]
    m = m_ref[...]
    x1 = jnp.where(m > 0.0, xt_ref[...], y1)
    ft2 = jnp.concatenate([x1, m, h2, lab_ref[...]], axis=1)
    xlf_o[...] = _dot(ft2, fwl_ref)
    xrf_o[...] = _dot(ft2, fwr_ref)


def _final_nextproj_body(accf_ref, denf_ref, fb_ref, h_ref, w2_ref, b2_ref,
                         m_ref, xt_ref, lab_ref, rwl_ref, rwr_ref, uwl_ref,
                         uwr_ref, x2_o, xlr_o, xrr_o, xlu_o, xru_o):
    sval = jnp.maximum(_mergevals(accf_ref, denf_ref, fb_ref[...]), 0.0)
    h = h_ref[...]
    y2 = (jnp.dot(sval, w2_ref[:H], preferred_element_type=jnp.float32)
          + jnp.dot(h, w2_ref[H:], preferred_element_type=jnp.float32)
          + b2_ref[...])
    m = m_ref[...]
    x2 = jnp.where(m > 0.0, xt_ref[...], y2)
    x2_o[...] = x2
    hin = jnp.concatenate([x2, m, lab_ref[...], h], axis=1)
    xlr_o[...] = _dot(hin, rwl_ref)
    xrr_o[...] = _dot(hin, rwr_ref)
    xlu_o[...] = _dot(hin, uwl_ref)
    xru_o[...] = _dot(hin, uwr_ref)


def _acc_spec():
    return pl.BlockSpec((NC, _BLK, H), lambda i: (0, i, 0))


def _den_spec():
    return pl.BlockSpec((NC, _BLK, 1), lambda i: (0, i, 0))


def _row_spec(w):
    return pl.BlockSpec((_BLK, w), lambda i: (i, 0))


def _full_spec(a, b):
    return pl.BlockSpec((a, b), lambda i: (0, 0))


_GRID = (N // _BLK,)
_NH = jax.ShapeDtypeStruct((N, H), jnp.float32)
_NF = jax.ShapeDtypeStruct((N, F), jnp.float32)


def _proj_ru(x2, maskf, labels, hidden, rwl, rwr, uwl, uwr):
    return pl.pallas_call(
        _proj_ru_body,
        grid=_GRID,
        in_specs=[_row_spec(F), _row_spec(1), _row_spec(L), _row_spec(H)]
                 + [_full_spec(DIN, H)] * 4,
        out_specs=[_row_spec(H)] * 4,
        out_shape=[_NH] * 4,
    )(x2, maskf, labels, hidden, rwl, rwr, uwl, uwr)


def _gates_cellproj(accr, denr, accu, denu, rb, ub, hidden, x2, maskf,
                    labels, cwl, cwr):
    accr, accu = accr.reshape(NC, NP, H), accu.reshape(NC, NP, H)
    denr = denr.reshape(NC, NP, 1)
    denu = denu.reshape(NC, NP, 1)
    return pl.pallas_call(
        _gates_cellproj_body,
        grid=_GRID,
        in_specs=[_acc_spec(), _den_spec(), _acc_spec(), _den_spec(),
                  _full_spec(1, H), _full_spec(1, H), _row_spec(H),
                  _row_spec(F), _row_spec(1), _row_spec(L),
                  _full_spec(DIN, H), _full_spec(DIN, H)],
        out_specs=[_row_spec(H)] * 3,
        out_shape=[_NH] * 3,
    )(accr, denr, accu, denu, rb, ub, hidden, x2, maskf, labels, cwl, cwr)


def _cell_ft2proj(accc, denc, cb, update, hidden, w1, b1, maskf, xt, labels,
                  fwl, fwr):
    accc = accc.reshape(NC, NP, H)
    denc = denc.reshape(NC, NP, 1)
    return pl.pallas_call(
        _cell_ft2proj_body,
        grid=_GRID,
        in_specs=[_acc_spec(), _den_spec(), _full_spec(1, H), _row_spec(H),
                  _row_spec(H), _full_spec(H, F), _full_spec(1, F),
                  _row_spec(1), _row_spec(F), _row_spec(L),
                  _full_spec(DIN, H), _full_spec(DIN, H)],
        out_specs=[_row_spec(H)] * 3,
        out_shape=[_NH] * 3,
    )(accc, denc, cb, update, hidden, w1, b1, maskf, xt, labels, fwl, fwr)


def _final_nextproj(accf, denf, fb, hidden, w2, b2, maskf, xt, labels,
                    rwl, rwr, uwl, uwr):
    accf = accf.reshape(NC, NP, H)
    denf = denf.reshape(NC, NP, 1)
    return pl.pallas_call(
        _final_nextproj_body,
        grid=_GRID,
        in_specs=[_acc_spec(), _den_spec(), _full_spec(1, H), _row_spec(H),
                  _full_spec(2 * H, F), _full_spec(1, F), _row_spec(1),
                  _row_spec(F), _row_spec(L)] + [_full_spec(DIN, H)] * 4,
        out_specs=[_row_spec(F)] + [_row_spec(H)] * 4,
        out_shape=[_NF] + [_NH] * 4,
    )(accf, denf, fb, hidden, w2, b2, maskf, xt, labels, rwl, rwr, uwl, uwr)


def kernel(x, edge_index, mask, labels, edge_weight,
           reset_Wl, reset_Wr, reset_att, reset_b,
           update_Wl, update_Wr, update_att, update_b,
           cell_Wl, cell_Wr, cell_att, cell_b,
           final_Wl, final_Wr, final_att, final_b,
           W1, b1, W2, b2):
    src = edge_index[0]
    dst = edge_index[1]
    mask_b = mask[:, None]
    mask_f = mask_b.astype(jnp.float32)
    hidden = jnp.ones((N, H), dtype=jnp.float32)
    xm = jnp.where(mask_b[None, :, :], x, 0.0)
    x2 = xm[0]
    rb, ub = reset_b[None, :], update_b[None, :]
    cb, fb = cell_b[None, :], final_b[None, :]
    b1r, b2r = b1[None, :], b2[None, :]
    xl_r, xr_r, xl_u, xr_u = _proj_ru(x2, mask_f, labels, hidden,
                                      reset_Wl, reset_Wr, update_Wl, update_Wr)
    outs = []
    for t in range(T):
        acc_r, den_r = _sc_conv(xl_r, xr_r, src, dst, edge_weight, reset_att)
        acc_u, den_u = _sc_conv(xl_u, xr_u, src, dst, edge_weight, update_att)
        update, xl_c, xr_c = _gates_cellproj(
            acc_r, den_r, acc_u, den_u, rb, ub, hidden, x2, mask_f, labels,
            cell_Wl, cell_Wr)
        acc_c, den_c = _sc_conv(xl_c, xr_c, src, dst, edge_weight, cell_att)
        hidden, xl_f, xr_f = _cell_ft2proj(
            acc_c, den_c, cb, update, hidden, W1, b1r, mask_f, xm[t], labels,
            final_Wl, final_Wr)
        acc_f, den_f = _sc_conv(xl_f, xr_f, src, dst, edge_weight, final_att)
        x2, xl_r, xr_r, xl_u, xr_u = _final_nextproj(
            acc_f, den_f, fb, hidden, W2, b2r, mask_f, xm[t], labels,
            reset_Wl, reset_Wr, update_Wl, update_Wr)
        outs.append(x2)
    return jnp.stack(outs)
